# baseline (speedup 1.0000x reference)
import functools

import jax
import jax.numpy as jnp
from jax import lax
from jax.experimental import pallas as pl
from jax.experimental.pallas import tpu as pltpu
from jax.experimental.pallas import tpu_sc as plsc

N_NODES = 100000
DIM = 128
NUM_SEG = 512
LANES = 16

ROWS = 80
NBLK = N_NODES // ROWS
NWORK = 32
BLK_PER_W = NBLK // NWORK
EXTRA_BLKS = NBLK - BLK_PER_W * NWORK
MAXB = BLK_PER_W + 1
ACC_ROWS = 1152
SL = ACC_ROWS // 16
CNT_W = 128


def _pool_body(out_hbm, z_hbm, batch_hbm, zsum_hbm, ones_hbm,
               sums_hbm, cnts_hbm,
               rows_a, rows_b, zbuf, bbuf, idx_v, ones_v,
               acc_sh, cnt_sh, sem_a, sem_b, sem_r, sem_o):
  c = lax.axis_index("c")
  s = lax.axis_index("s")
  wid = c * 16 + s

  pltpu.sync_copy(zsum_hbm.at[pl.ds(s * SL, SL), :], acc_sh.at[pl.ds(s * SL, SL), :])
  pltpu.sync_copy(zsum_hbm.at[pl.ds(s * SL, SL), :], cnt_sh.at[pl.ds(s * SL, SL), :])
  pltpu.sync_copy(ones_hbm, ones_v)
  plsc.subcore_barrier()

  nblk = BLK_PER_W + jnp.where(wid < EXTRA_BLKS, 1, 0)
  blk0 = wid * BLK_PER_W + jnp.minimum(wid, EXTRA_BLKS)
  row0 = blk0 * ROWS

  base_rows = BLK_PER_W * ROWS
  pltpu.sync_copy(z_hbm.at[pl.ds(row0, base_rows)], zbuf.at[pl.ds(0, base_rows)])
  pltpu.sync_copy(batch_hbm.at[pl.ds(row0, base_rows)], bbuf.at[pl.ds(0, base_rows)])

  @pl.when(nblk == MAXB)
  def _():
    pltpu.sync_copy(z_hbm.at[pl.ds(row0 + base_rows, ROWS)],
                    zbuf.at[pl.ds(base_rows, ROWS)])
    pltpu.sync_copy(batch_hbm.at[pl.ds(row0 + base_rows, ROWS)],
                    bbuf.at[pl.ds(base_rows, ROWS)])

  pltpu.async_copy(out_hbm.at[pl.ds(row0, ROWS), :], rows_a, sem_a)

  def process(g, cur_buf, cur_sem, nxt_buf, nxt_sem):
    grow = row0 + g * ROWS
    pltpu.make_async_copy(out_hbm.at[pl.ds(grow, ROWS), :], cur_buf, cur_sem).wait()

    @pl.when(g + 1 < nblk)
    def _():
      pltpu.async_copy(out_hbm.at[pl.ds(grow + ROWS, ROWS), :], nxt_buf, nxt_sem)

    for i in range(ROWS // LANES):
      off = g * ROWS + i * LANES
      bv = bbuf[pl.ds(off, LANES)]
      zv = zbuf[pl.ds(off, LANES)]
      idx = bv + jnp.where(zv == 100, NUM_SEG, 0).astype(jnp.int32)
      idx_v[pl.ds(i * LANES, LANES)] = idx
    d1 = pltpu.async_copy(cur_buf, acc_sh.at[idx_v], sem_r, add=True)
    d2 = pltpu.async_copy(ones_v, cnt_sh.at[idx_v], sem_o, add=True)
    d1.wait()
    d2.wait()

  def step(g, carry):
    @pl.when(g % 2 == 0)
    def _():
      process(g, rows_a, sem_a, rows_b, sem_b)

    @pl.when(g % 2 == 1)
    def _():
      process(g, rows_b, sem_b, rows_a, sem_a)

    return carry

  lax.fori_loop(0, nblk, step, 0)

  plsc.subcore_barrier()
  pltpu.sync_copy(acc_sh.at[pl.ds(s * SL, SL), :], sums_hbm.at[c, pl.ds(s * SL, SL), :])
  pltpu.sync_copy(cnt_sh.at[pl.ds(s * SL, SL), :], cnts_hbm.at[c, pl.ds(s * SL, SL), :])


@jax.jit
def _pool(out, z, batch):
  zsum = jnp.zeros((ACC_ROWS, DIM), jnp.float32)
  ones = jnp.ones((ROWS, CNT_W), jnp.float32)
  mesh = plsc.VectorSubcoreMesh(core_axis_name="c", subcore_axis_name="s")
  f = pl.kernel(
      _pool_body,
      out_type=(
          jax.ShapeDtypeStruct((2, ACC_ROWS, DIM), jnp.float32),
          jax.ShapeDtypeStruct((2, ACC_ROWS, CNT_W), jnp.float32),
      ),
      mesh=mesh,
      scratch_types=[
          pltpu.VMEM((ROWS, DIM), jnp.float32),
          pltpu.VMEM((ROWS, DIM), jnp.float32),
          pltpu.VMEM((MAXB * ROWS,), jnp.int32),
          pltpu.VMEM((MAXB * ROWS,), jnp.int32),
          pltpu.VMEM((ROWS,), jnp.int32),
          pltpu.VMEM((ROWS, CNT_W), jnp.float32),
          pltpu.VMEM_SHARED((ACC_ROWS, DIM), jnp.float32),
          pltpu.VMEM_SHARED((ACC_ROWS, CNT_W), jnp.float32),
          pltpu.SemaphoreType.DMA,
          pltpu.SemaphoreType.DMA,
          pltpu.SemaphoreType.DMA,
          pltpu.SemaphoreType.DMA,
      ],
  )
  return f(out, z, batch, zsum, ones)


def _mlp_body(sums_ref, cnts_ref, w1_ref, b1_ref, w2_ref, o_ref):
  ssum = sums_ref[0] + sums_ref[1]
  cnt = cnts_ref[0, :, 0:1] + cnts_ref[1, :, 0:1]
  real_s = ssum[0:NUM_SEG]
  virt_s = ssum[NUM_SEG:2 * NUM_SEG]
  cr = cnt[0:NUM_SEG]
  cv = cnt[NUM_SEG:2 * NUM_SEG]
  real_m = jnp.where(cr > 0, real_s / jnp.maximum(cr, 1.0), 0.0)
  virt_m = jnp.where(cv > 0, virt_s / jnp.maximum(cv, 1.0), 0.0)
  w1 = w1_ref[...]
  b1 = b1_ref[...]
  w2 = w2_ref[...]
  h_r = jnp.tanh(jnp.dot(real_m, w1, precision=lax.Precision.HIGHEST) + b1)
  h_v = jnp.tanh(jnp.dot(virt_m, w1, precision=lax.Precision.HIGHEST) + b1)
  s_r = jnp.sum(h_r * w2, axis=1, keepdims=True)
  s_v = jnp.sum(h_v * w2, axis=1, keepdims=True)
  m = jnp.maximum(s_r, s_v)
  er = jnp.exp(s_r - m)
  ev = jnp.exp(s_v - m)
  wr = er / (er + ev)
  o_ref[...] = wr * real_m + (1.0 - wr) * virt_m


@jax.jit
def kernel(out, z, batch, W1, b1, W2):
  sums, cnts = _pool(out, z, batch)
  b1r = b1.reshape(1, DIM)
  w2r = W2.reshape(DIM, 1).T
  return pl.pallas_call(
      _mlp_body,
      out_shape=jax.ShapeDtypeStruct((NUM_SEG, DIM), jnp.float32),
  )(sums, cnts, W1, b1r, w2r)


# speedup vs baseline: 1.0162x; 1.0162x over previous
import functools

import jax
import jax.numpy as jnp
from jax import lax
from jax.experimental import pallas as pl
from jax.experimental.pallas import tpu as pltpu
from jax.experimental.pallas import tpu_sc as plsc

N_NODES = 100000
DIM = 128
NUM_SEG = 512
LANES = 16

ROWS = 80
NBLK = N_NODES // ROWS
NWORK = 32
BLK_PER_W = NBLK // NWORK
EXTRA_BLKS = NBLK - BLK_PER_W * NWORK
MAXB = BLK_PER_W + 1
ACC_ROWS = 1152
SL = ACC_ROWS // 16
CNT_W = 128


def _pool_body(out_hbm, z_hbm, batch_hbm, zsum_hbm, ones_hbm,
               sums_hbm, cnts_hbm,
               rows_a, rows_b, zbuf, bbuf, idx_a, idx_b, ones_v,
               acc_sh, cnt_sh, sem_a, sem_b, sr_a, sr_b, so_a, so_b):
  c = lax.axis_index("c")
  s = lax.axis_index("s")
  wid = c * 16 + s

  pltpu.sync_copy(zsum_hbm.at[pl.ds(s * SL, SL), :], acc_sh.at[pl.ds(s * SL, SL), :])
  pltpu.sync_copy(zsum_hbm.at[pl.ds(s * SL, SL), :], cnt_sh.at[pl.ds(s * SL, SL), :])
  pltpu.sync_copy(ones_hbm, ones_v)
  plsc.subcore_barrier()

  nblk = BLK_PER_W + jnp.where(wid < EXTRA_BLKS, 1, 0)
  blk0 = wid * BLK_PER_W + jnp.minimum(wid, EXTRA_BLKS)
  row0 = blk0 * ROWS

  base_rows = BLK_PER_W * ROWS
  pltpu.sync_copy(z_hbm.at[pl.ds(row0, base_rows)], zbuf.at[pl.ds(0, base_rows)])
  pltpu.sync_copy(batch_hbm.at[pl.ds(row0, base_rows)], bbuf.at[pl.ds(0, base_rows)])

  @pl.when(nblk == MAXB)
  def _():
    pltpu.sync_copy(z_hbm.at[pl.ds(row0 + base_rows, ROWS)],
                    zbuf.at[pl.ds(base_rows, ROWS)])
    pltpu.sync_copy(batch_hbm.at[pl.ds(row0 + base_rows, ROWS)],
                    bbuf.at[pl.ds(base_rows, ROWS)])

  pltpu.async_copy(out_hbm.at[pl.ds(row0, ROWS), :], rows_a, sem_a)

  def process(g, cur_buf, cur_gsem, cur_idx, cur_sr, cur_so,
              oth_buf, oth_gsem, oth_idx, oth_sr, oth_so):
    grow = row0 + g * ROWS
    pltpu.make_async_copy(out_hbm.at[pl.ds(grow, ROWS), :], cur_buf, cur_gsem).wait()
    for i in range(ROWS // LANES):
      off = g * ROWS + i * LANES
      bv = bbuf[pl.ds(off, LANES)]
      zv = zbuf[pl.ds(off, LANES)]
      idx = bv + jnp.where(zv == 100, NUM_SEG, 0).astype(jnp.int32)
      cur_idx[pl.ds(i * LANES, LANES)] = idx
    pltpu.async_copy(cur_buf, acc_sh.at[cur_idx], cur_sr, add=True)
    pltpu.async_copy(ones_v, cnt_sh.at[cur_idx], cur_so, add=True)

    @pl.when(g >= 1)
    def _():
      pltpu.make_async_copy(oth_buf, acc_sh.at[oth_idx], oth_sr).wait()
      pltpu.make_async_copy(ones_v, cnt_sh.at[oth_idx], oth_so).wait()

    @pl.when(g + 1 < nblk)
    def _():
      pltpu.async_copy(out_hbm.at[pl.ds(grow + ROWS, ROWS), :], oth_buf, oth_gsem)

  def step(g, carry):
    @pl.when(g % 2 == 0)
    def _():
      process(g, rows_a, sem_a, idx_a, sr_a, so_a,
              rows_b, sem_b, idx_b, sr_b, so_b)

    @pl.when(g % 2 == 1)
    def _():
      process(g, rows_b, sem_b, idx_b, sr_b, so_b,
              rows_a, sem_a, idx_a, sr_a, so_a)

    return carry

  lax.fori_loop(0, nblk, step, 0)

  @pl.when((nblk - 1) % 2 == 0)
  def _():
    pltpu.make_async_copy(rows_a, acc_sh.at[idx_a], sr_a).wait()
    pltpu.make_async_copy(ones_v, cnt_sh.at[idx_a], so_a).wait()

  @pl.when((nblk - 1) % 2 == 1)
  def _():
    pltpu.make_async_copy(rows_b, acc_sh.at[idx_b], sr_b).wait()
    pltpu.make_async_copy(ones_v, cnt_sh.at[idx_b], so_b).wait()

  plsc.subcore_barrier()
  pltpu.sync_copy(acc_sh.at[pl.ds(s * SL, SL), :], sums_hbm.at[c, pl.ds(s * SL, SL), :])
  pltpu.sync_copy(cnt_sh.at[pl.ds(s * SL, SL), :], cnts_hbm.at[c, pl.ds(s * SL, SL), :])


@jax.jit
def _pool(out, z, batch):
  zsum = jnp.zeros((ACC_ROWS, DIM), jnp.float32)
  ones = jnp.ones((ROWS, CNT_W), jnp.float32)
  mesh = plsc.VectorSubcoreMesh(core_axis_name="c", subcore_axis_name="s")
  f = pl.kernel(
      _pool_body,
      out_type=(
          jax.ShapeDtypeStruct((2, ACC_ROWS, DIM), jnp.float32),
          jax.ShapeDtypeStruct((2, ACC_ROWS, CNT_W), jnp.float32),
      ),
      mesh=mesh,
      scratch_types=[
          pltpu.VMEM((ROWS, DIM), jnp.float32),
          pltpu.VMEM((ROWS, DIM), jnp.float32),
          pltpu.VMEM((MAXB * ROWS,), jnp.int32),
          pltpu.VMEM((MAXB * ROWS,), jnp.int32),
          pltpu.VMEM((ROWS,), jnp.int32),
          pltpu.VMEM((ROWS,), jnp.int32),
          pltpu.VMEM((ROWS, CNT_W), jnp.float32),
          pltpu.VMEM_SHARED((ACC_ROWS, DIM), jnp.float32),
          pltpu.VMEM_SHARED((ACC_ROWS, CNT_W), jnp.float32),
          pltpu.SemaphoreType.DMA,
          pltpu.SemaphoreType.DMA,
          pltpu.SemaphoreType.DMA,
          pltpu.SemaphoreType.DMA,
          pltpu.SemaphoreType.DMA,
          pltpu.SemaphoreType.DMA,
      ],
  )
  return f(out, z, batch, zsum, ones)


def _mlp_body(sums_ref, cnts_ref, w1_ref, b1_ref, w2_ref, o_ref):
  ssum = sums_ref[0] + sums_ref[1]
  cnt = cnts_ref[0, :, 0:1] + cnts_ref[1, :, 0:1]
  real_s = ssum[0:NUM_SEG]
  virt_s = ssum[NUM_SEG:2 * NUM_SEG]
  cr = cnt[0:NUM_SEG]
  cv = cnt[NUM_SEG:2 * NUM_SEG]
  real_m = jnp.where(cr > 0, real_s / jnp.maximum(cr, 1.0), 0.0)
  virt_m = jnp.where(cv > 0, virt_s / jnp.maximum(cv, 1.0), 0.0)
  w1 = w1_ref[...]
  b1 = b1_ref[...]
  w2 = w2_ref[...]
  h_r = jnp.tanh(jnp.dot(real_m, w1, precision=lax.Precision.HIGHEST) + b1)
  h_v = jnp.tanh(jnp.dot(virt_m, w1, precision=lax.Precision.HIGHEST) + b1)
  s_r = jnp.sum(h_r * w2, axis=1, keepdims=True)
  s_v = jnp.sum(h_v * w2, axis=1, keepdims=True)
  m = jnp.maximum(s_r, s_v)
  er = jnp.exp(s_r - m)
  ev = jnp.exp(s_v - m)
  wr = er / (er + ev)
  o_ref[...] = wr * real_m + (1.0 - wr) * virt_m


@jax.jit
def kernel(out, z, batch, W1, b1, W2):
  sums, cnts = _pool(out, z, batch)
  b1r = b1.reshape(1, DIM)
  w2r = W2.reshape(DIM, 1).T
  return pl.pallas_call(
      _mlp_body,
      out_shape=jax.ShapeDtypeStruct((NUM_SEG, DIM), jnp.float32),
  )(sums, cnts, W1, b1r, w2r)
